# baseline (speedup 1.0000x reference)
import functools

import jax
import jax.numpy as jnp
from jax import lax
from jax.experimental import pallas as pl
from jax.experimental.pallas import tpu as pltpu
from jax.experimental.pallas import tpu_sc as plsc

_NC = 2
_NS = 16
_NW = _NC * _NS
_LANES = 16
_GCHUNK = 128


@functools.lru_cache(maxsize=None)
def _gather_mean(V, D, B, CTX):
    n_idx = (B * CTX) // _NW
    rows_per_w = B // _NW
    n_ch = n_idx // _GCHUNK
    d_lanes = D // _LANES
    scale = 1.0 / CTX
    mesh = plsc.VectorSubcoreMesh(core_axis_name="c", subcore_axis_name="s")

    @functools.partial(
        pl.kernel,
        mesh=mesh,
        compiler_params=pltpu.CompilerParams(use_tc_tiling_on_sc=False),
        out_type=jax.ShapeDtypeStruct((B, D), jnp.float32),
        scratch_types=[
            pltpu.VMEM((n_ch, _GCHUNK), jnp.int32),
            pltpu.VMEM((n_idx, D), jnp.float32),
            pltpu.VMEM((rows_per_w, D), jnp.float32),
            pltpu.SemaphoreType.DMA,
        ],
    )
    def k(idx_hbm, table_hbm, out_hbm, idx_v, rows_v, out_v, sem):
        wid = lax.axis_index("s") * _NC + lax.axis_index("c")
        pltpu.sync_copy(idx_hbm.at[wid], idx_v)
        copies = [
            pltpu.async_copy(
                table_hbm.at[idx_v.at[j]],
                rows_v.at[pl.ds(j * _GCHUNK, _GCHUNK)],
                sem,
            )
            for j in range(n_ch)
        ]
        for cp in copies:
            cp.wait()

        def row_body(i, carry):
            rbase = i * CTX
            for dl in range(d_lanes):
                sl = pl.ds(dl * _LANES, _LANES)
                acc = rows_v[rbase, sl]
                for c in range(1, CTX):
                    acc = acc + rows_v[rbase + c, sl]
                out_v[i, sl] = acc * scale
            return carry

        lax.fori_loop(0, rows_per_w, row_body, 0)
        pltpu.sync_copy(out_v, out_hbm.at[pl.ds(wid * rows_per_w, rows_per_w)])

    return k


def _mm_body(h_ref, w_ref, b_ref, o_ref):
    o_ref[...] = (
        lax.dot_general(
            h_ref[...], w_ref[...],
            (((1,), (1,)), ((), ())),
            preferred_element_type=jnp.float32,
        )
        + b_ref[...]
    )


@functools.lru_cache(maxsize=None)
def _matmul(B, D, V, vt):
    grid = (pl.cdiv(V, vt),)
    return pl.pallas_call(
        _mm_body,
        grid=grid,
        in_specs=[
            pl.BlockSpec((B, D), lambda j: (0, 0)),
            pl.BlockSpec((vt, D), lambda j: (j, 0)),
            pl.BlockSpec((1, vt), lambda j: (0, j)),
        ],
        out_specs=pl.BlockSpec((B, vt), lambda j: (0, j)),
        out_shape=jax.ShapeDtypeStruct((B, V), jnp.float32),
    )


def kernel(x, emb_table, W, b):
    B, CTX = x.shape
    V, D = emb_table.shape
    idx3d = x.astype(jnp.int32).reshape(_NW, -1, _GCHUNK)
    h = _gather_mean(V, D, B, CTX)(idx3d, emb_table)
    logits = _matmul(B, D, V, 2048)(h, W, b.reshape(1, V))
    return logits


# speedup vs baseline: 2.7394x; 2.7394x over previous
import functools

import jax
import jax.numpy as jnp
from jax import lax
from jax.experimental import pallas as pl
from jax.experimental.pallas import tpu as pltpu
from jax.experimental.pallas import tpu_sc as plsc

_NC = 2
_NS = 16
_NW = _NC * _NS
_LANES = 16
_GCHUNK = 128


@functools.lru_cache(maxsize=None)
def _gather_mean(V, D, B, CTX):
    n_idx = (B * CTX) // _NW
    rows_per_w = B // _NW
    n_ch = n_idx // _GCHUNK
    d_lanes = D // _LANES
    scale = 1.0 / CTX
    mesh = plsc.VectorSubcoreMesh(core_axis_name="c", subcore_axis_name="s")

    @functools.partial(
        pl.kernel,
        mesh=mesh,
        compiler_params=pltpu.CompilerParams(
            use_tc_tiling_on_sc=False, needs_layout_passes=False),
        out_type=jax.ShapeDtypeStruct((D, B), jnp.float32),
        scratch_types=[
            pltpu.VMEM((n_ch, _GCHUNK), jnp.int32),
            pltpu.VMEM((n_idx, D), jnp.float32),
            pltpu.VMEM((D, rows_per_w), jnp.float32),
            pltpu.SemaphoreType.DMA,
        ],
    )
    def k(idx_hbm, table_hbm, out_hbm, idx_v, rows_v, out_v, sem):
        wid = lax.axis_index("s") * _NC + lax.axis_index("c")
        pltpu.sync_copy(idx_hbm.at[wid], idx_v)
        copies = [
            pltpu.async_copy(
                table_hbm.at[idx_v.at[j]],
                rows_v.at[pl.ds(j * _GCHUNK, _GCHUNK)],
                sem,
            )
            for j in range(n_ch)
        ]
        for cp in copies:
            cp.wait()

        lane_iota = lax.iota(jnp.int32, _LANES)

        def row_body(i, carry):
            rbase = i * CTX
            col = jnp.full((_LANES,), 0, jnp.int32) + i
            for dl in range(d_lanes):
                sl = pl.ds(dl * _LANES, _LANES)
                acc = rows_v[rbase, sl]
                for c in range(1, CTX):
                    acc = acc + rows_v[rbase + c, sl]
                plsc.store_scatter(
                    out_v, [dl * _LANES + lane_iota, col], acc * scale)
            return carry

        lax.fori_loop(0, rows_per_w, row_body, 0)
        pltpu.sync_copy(out_v, out_hbm.at[:, pl.ds(wid * rows_per_w, rows_per_w)])

    return k


def _mm_body(w_ref, b_ref, h_ref, o_ref):
    lhs = jnp.concatenate([w_ref[...], b_ref[...]], axis=0)
    o_ref[...] = lax.dot_general(
        lhs, h_ref[...],
        (((0,), (0,)), ((), ())),
        preferred_element_type=jnp.float32,
    )


@functools.lru_cache(maxsize=None)
def _matmul(B, D, V, vt):
    grid = (pl.cdiv(V, vt),)
    return pl.pallas_call(
        _mm_body,
        grid=grid,
        in_specs=[
            pl.BlockSpec((D, vt), lambda j: (0, j)),
            pl.BlockSpec((1, vt), lambda j: (0, j)),
            pl.BlockSpec((D + 1, B), lambda j: (0, 0)),
        ],
        out_specs=pl.BlockSpec((vt, B), lambda j: (j, 0)),
        out_shape=jax.ShapeDtypeStruct((V, B), jnp.float32),
    )


def kernel(x, emb_table, W, b):
    B, CTX = x.shape
    V, D = emb_table.shape
    idx3d = x.astype(jnp.int32).reshape(_NW, -1, _GCHUNK)
    h_t = _gather_mean(V, D, B, CTX)(idx3d, emb_table)
    h_aug = jnp.concatenate([h_t, jnp.ones((1, B), jnp.float32)], axis=0)
    out_t = _matmul(B, D, V, 2048)(W.T, b.reshape(1, V), h_aug)
    return out_t.T


# speedup vs baseline: 2.9834x; 1.0891x over previous
import functools

import jax
import jax.numpy as jnp
from jax import lax
from jax.experimental import pallas as pl
from jax.experimental.pallas import tpu as pltpu
from jax.experimental.pallas import tpu_sc as plsc

_NC = 2
_NS = 16
_NW = _NC * _NS
_LANES = 16
_GCHUNK = 128


@functools.lru_cache(maxsize=None)
def _gather_mean(V2, D, B, CTX):
    n_idx = (B * CTX) // _NW
    rows_per_w = B // _NW
    n_ch = n_idx // _GCHUNK
    d_lanes = D // _LANES
    scale = 1.0 / CTX
    mesh = plsc.VectorSubcoreMesh(core_axis_name="c", subcore_axis_name="s")

    @functools.partial(
        pl.kernel,
        mesh=mesh,
        compiler_params=pltpu.CompilerParams(
            use_tc_tiling_on_sc=False, needs_layout_passes=False),
        out_type=jax.ShapeDtypeStruct((D, B), jnp.float32),
        scratch_types=[
            pltpu.VMEM((n_ch, _GCHUNK), jnp.int32),
            pltpu.VMEM((n_idx, D), jnp.float32),
            pltpu.VMEM((D, rows_per_w), jnp.float32),
            pltpu.SemaphoreType.DMA,
        ],
    )
    def k(idx_hbm, table_hbm, out_hbm, idx_v, rows_v, out_v, sem):
        wid = lax.axis_index("s") * _NC + lax.axis_index("c")
        pltpu.sync_copy(idx_hbm.at[wid], idx_v)
        copies = [
            pltpu.async_copy(
                table_hbm.at[idx_v.at[j]],
                rows_v.at[pl.ds(j * _GCHUNK, _GCHUNK)],
                sem,
            )
            for j in range(n_ch)
        ]
        for cp in copies:
            cp.wait()

        lane_iota = lax.iota(jnp.int32, _LANES)

        def row_body(i, carry):
            rbase = i * CTX
            col = jnp.full((_LANES,), 0, jnp.int32) + i
            for dl in range(d_lanes):
                sl = pl.ds(dl * _LANES, _LANES)
                acc = rows_v[rbase, sl]
                for c in range(1, CTX):
                    acc = acc + rows_v[rbase + c, sl]
                plsc.store_scatter(
                    out_v, [dl * _LANES + lane_iota, col], acc * scale)
            return carry

        lax.fori_loop(0, rows_per_w, row_body, 0)
        pltpu.sync_copy(out_v, out_hbm.at[:, pl.ds(wid * rows_per_w, rows_per_w)])

    return k


def _fmt_body(x0_ref, x1_ref, o_ref):
    d = x0_ref.shape[0]
    o_ref[:, 0:d] = lax.transpose(x0_ref[...], (1, 0))
    o_ref[:, d:2 * d] = lax.transpose(x1_ref[...], (1, 0))


@functools.lru_cache(maxsize=None)
def _fmt_table(V, D, cr):
    n_grp = pl.cdiv(V, 2 * cr)
    return pl.pallas_call(
        _fmt_body,
        grid=(n_grp,),
        in_specs=[
            pl.BlockSpec((D, cr), lambda j: (0, 2 * j)),
            pl.BlockSpec((D, cr), lambda j: (0, 2 * j + 1)),
        ],
        out_specs=pl.BlockSpec((cr, 2 * D), lambda j: (j, 0)),
        out_shape=jax.ShapeDtypeStruct((n_grp * cr, 2 * D), jnp.float32),
    )


def _mm_body(w_ref, b_ref, h_ref, o_ref):
    lhs = jnp.concatenate([w_ref[...], b_ref[...]], axis=0)
    o_ref[...] = lax.dot_general(
        lhs, h_ref[...],
        (((0,), (0,)), ((), ())),
        preferred_element_type=jnp.float32,
    )


@functools.lru_cache(maxsize=None)
def _matmul(B, D, V, vt):
    grid = (pl.cdiv(V, vt),)
    return pl.pallas_call(
        _mm_body,
        grid=grid,
        in_specs=[
            pl.BlockSpec((D, vt), lambda j: (0, j)),
            pl.BlockSpec((1, vt), lambda j: (0, j)),
            pl.BlockSpec((D + 1, B), lambda j: (0, 0)),
        ],
        out_specs=pl.BlockSpec((vt, B), lambda j: (j, 0)),
        out_shape=jax.ShapeDtypeStruct((V, B), jnp.float32),
    )


def kernel(x, emb_table, W, b):
    B, CTX = x.shape
    V, D = emb_table.shape
    cr = 1024
    xi = x.astype(jnp.int32)
    m = xi // cr
    lo = xi - m * cr
    xk = (m & ~1) * cr + 2 * lo + (m & 1)
    idx3d = xk.reshape(_NW, -1, _GCHUNK)
    table_lin = _fmt_table(V, D, cr)(emb_table.T, emb_table.T)
    table_lin = table_lin.reshape(table_lin.shape[0] * 2, D)
    h_t = _gather_mean(table_lin.shape[0], D, B, CTX)(idx3d, table_lin)
    h_aug = jnp.concatenate([h_t, jnp.ones((1, B), jnp.float32)], axis=0)
    out_t = _matmul(B, D, V, 4096)(W.T, b.reshape(1, V), h_aug)
    return out_t.T


# speedup vs baseline: 2.9859x; 1.0008x over previous
import functools

import jax
import jax.numpy as jnp
from jax import lax
from jax.experimental import pallas as pl
from jax.experimental.pallas import tpu as pltpu
from jax.experimental.pallas import tpu_sc as plsc

_NC = 2
_NS = 16
_NW = _NC * _NS
_LANES = 16
_GCHUNK = 128


@functools.lru_cache(maxsize=None)
def _gather_mean(V2, D, B, CTX):
    n_idx = (B * CTX) // _NW
    rows_per_w = B // _NW
    n_ch = n_idx // _GCHUNK
    d_lanes = D // _LANES
    scale = 1.0 / CTX
    mesh = plsc.VectorSubcoreMesh(core_axis_name="c", subcore_axis_name="s")

    @functools.partial(
        pl.kernel,
        mesh=mesh,
        compiler_params=pltpu.CompilerParams(
            use_tc_tiling_on_sc=False, needs_layout_passes=False),
        out_type=jax.ShapeDtypeStruct((D, B), jnp.float32),
        scratch_types=[
            pltpu.VMEM((n_ch, _GCHUNK), jnp.int32),
            pltpu.VMEM((n_idx, D), jnp.float32),
            pltpu.VMEM((D, rows_per_w), jnp.float32),
            pltpu.SemaphoreType.DMA,
        ],
    )
    def k(idx_hbm, table_hbm, out_hbm, idx_v, rows_v, out_v, sem):
        wid = lax.axis_index("s") * _NC + lax.axis_index("c")
        pltpu.sync_copy(idx_hbm.at[wid], idx_v)
        copies = [
            pltpu.async_copy(
                table_hbm.at[idx_v.at[j]],
                rows_v.at[pl.ds(j * _GCHUNK, _GCHUNK)],
                sem,
            )
            for j in range(n_ch)
        ]
        for cp in copies:
            cp.wait()

        lane_iota = lax.iota(jnp.int32, _LANES)

        def row_body(i, carry):
            rbase = i * CTX
            col = jnp.full((_LANES,), 0, jnp.int32) + i
            for dl in range(d_lanes):
                sl = pl.ds(dl * _LANES, _LANES)
                acc = rows_v[rbase, sl]
                for c in range(1, CTX):
                    acc = acc + rows_v[rbase + c, sl]
                plsc.store_scatter(
                    out_v, [dl * _LANES + lane_iota, col], acc * scale)
            return carry

        lax.fori_loop(0, rows_per_w, row_body, 0)
        pltpu.sync_copy(out_v, out_hbm.at[:, pl.ds(wid * rows_per_w, rows_per_w)])

    return k


def _fmt_body(x0_ref, x1_ref, o_ref):
    d = x0_ref.shape[0]
    o_ref[:, 0:d] = lax.transpose(x0_ref[...], (1, 0))
    o_ref[:, d:2 * d] = lax.transpose(x1_ref[...], (1, 0))


@functools.lru_cache(maxsize=None)
def _fmt_table(V, D, cr):
    n_grp = pl.cdiv(V, 2 * cr)
    return pl.pallas_call(
        _fmt_body,
        grid=(n_grp,),
        in_specs=[
            pl.BlockSpec((D, cr), lambda j: (0, 2 * j)),
            pl.BlockSpec((D, cr), lambda j: (0, 2 * j + 1)),
        ],
        out_specs=pl.BlockSpec((cr, 2 * D), lambda j: (j, 0)),
        out_shape=jax.ShapeDtypeStruct((n_grp * cr, 2 * D), jnp.float32),
    )


def _mm_body(w_ref, b_ref, h_ref, o_ref):
    lhs = jnp.concatenate([w_ref[...], b_ref[...]], axis=0)
    o_ref[...] = lax.dot_general(
        lhs, h_ref[...],
        (((0,), (0,)), ((), ())),
        preferred_element_type=jnp.float32,
    )


@functools.lru_cache(maxsize=None)
def _matmul(B, D, V, vt):
    grid = (pl.cdiv(V, vt),)
    return pl.pallas_call(
        _mm_body,
        grid=grid,
        in_specs=[
            pl.BlockSpec((D, vt), lambda j: (0, j)),
            pl.BlockSpec((1, vt), lambda j: (0, j)),
            pl.BlockSpec((D + 1, B), lambda j: (0, 0)),
        ],
        out_specs=pl.BlockSpec((vt, B), lambda j: (j, 0)),
        out_shape=jax.ShapeDtypeStruct((V, B), jnp.float32),
    )


def kernel(x, emb_table, W, b):
    B, CTX = x.shape
    V, D = emb_table.shape
    cr = 1024
    xi = x.astype(jnp.int32)
    m = xi // cr
    lo = xi - m * cr
    xk = (m & ~1) * cr + 2 * lo + (m & 1)
    idx3d = xk.reshape(_NW, -1, _GCHUNK)
    table_lin = _fmt_table(V, D, cr)(emb_table.T, emb_table.T)
    table_lin = table_lin.reshape(table_lin.shape[0] * 2, D)
    h_t = _gather_mean(table_lin.shape[0], D, B, CTX)(idx3d, table_lin)
    h_aug = jnp.concatenate([h_t, jnp.ones((1, B), jnp.float32)], axis=0)
    out_t = _matmul(B, D, V, 5120)(W.T, b.reshape(1, V), h_aug)
    return out_t.T
